# baseline (speedup 1.0000x reference)
import dataclasses
import functools

import jax
import jax.numpy as jnp
from jax import lax
from jax.experimental import pallas as pl
from jax.experimental.pallas import tpu as pltpu
from jax.experimental.pallas import tpu_sc as plsc

N = 10000
E = 160000
F = 256
H = 512
FH = 128
NC, NS = 2, 16
NPAD = 10112
HALF = NPAD // 2
ACCR = 5248
ZPT = ACCR // NS
RPT = NPAD // NS
EPT = E // NS
CH = (EPT + 127) // 128
EPTD = E // (NC * NS)
DCH = (EPTD + 127) // 128
BN = 1000

_mesh = plsc.VectorSubcoreMesh(core_axis_name="c", subcore_axis_name="s")

_cp = pltpu.CompilerParams()
if "needs_layout_passes" in pltpu.CompilerParams.__dataclass_fields__:
    _cp = dataclasses.replace(_cp, needs_layout_passes=False)


@functools.partial(
    pl.kernel,
    out_type=jax.ShapeDtypeStruct((NC, NPAD, 8), jnp.float32),
    mesh=_mesh,
    scratch_types=[
        pltpu.VMEM((DCH, 128), jnp.int32),
        pltpu.VMEM((128, 8), jnp.float32),
        pltpu.VMEM_SHARED((NPAD, 8), jnp.float32),
    ],
)
def _deg_kernel(dsti_hbm, ones_hbm, zeros_hbm, out_hbm, idx_v, ones_v, acc_sh):
    c = lax.axis_index("c")
    s = lax.axis_index("s")
    r0 = s * RPT
    pltpu.sync_copy(zeros_hbm.at[pl.ds(r0, RPT)], acc_sh.at[pl.ds(r0, RPT)])
    pltpu.sync_copy(ones_hbm, ones_v)
    pltpu.sync_copy(dsti_hbm.at[c, s], idx_v)
    plsc.subcore_barrier()

    @pl.loop(0, DCH)
    def _(j):
        pltpu.sync_copy(ones_v, acc_sh.at[idx_v.at[j]], add=True)

    plsc.subcore_barrier()
    pltpu.sync_copy(acc_sh.at[pl.ds(r0, RPT)], out_hbm.at[c, pl.ds(r0, RPT)])


CHE = CH * 128


@functools.partial(
    pl.kernel,
    out_type=jax.ShapeDtypeStruct((NC, NPAD, FH), jnp.float32),
    mesh=_mesh,
    scratch_types=[
        pltpu.VMEM((CHE + 128,), jnp.int32),
        pltpu.VMEM((CHE + 128,), jnp.int32),
        pltpu.VMEM((CHE + 128,), jnp.int32),
        pltpu.VMEM((CHE + 128,), jnp.int32),
        pltpu.VMEM((128, FH), jnp.float32),
        pltpu.VMEM((128, FH), jnp.float32),
        pltpu.VMEM_SHARED((ACCR, FH), jnp.float32),
        pltpu.SemaphoreType.DMA,
        pltpu.SemaphoreType.DMA,
    ],
    compiler_params=_cp,
)
def _agg_kernel(y_hbm, srcf_hbm, dstf_hbm, zeros_hbm,
                out_hbm, sa_v, da_v, sb_v, db_v, ra_v, rb_v,
                acc_sh, sem_a, sem_b):
    c = lax.axis_index("c")
    s = lax.axis_index("s")
    coff = c * N

    pltpu.sync_copy(srcf_hbm.at[s], sa_v.at[pl.ds(0, CHE)])
    pltpu.sync_copy(dstf_hbm.at[s], da_v.at[pl.ds(0, CHE)])

    def part_body(i, off):
        off_a, off_b = off
        sv = sa_v[pl.ds(i * 16, 16)] + coff
        dv = da_v[pl.ds(i * 16, 16)]
        m_a = dv < HALF
        m_b = jnp.logical_and(dv >= HALF, dv < N)
        plsc.store_compressed(sa_v.at[pl.ds(off_a, 16)], sv, mask=m_a)
        plsc.store_compressed(da_v.at[pl.ds(off_a, 16)], dv, mask=m_a)
        plsc.store_compressed(sb_v.at[pl.ds(off_b, 16)], sv, mask=m_b)
        plsc.store_compressed(db_v.at[pl.ds(off_b, 16)], dv - HALF, mask=m_b)
        n_a = jnp.max(plsc.all_reduce_population_count(m_a))
        n_b = jnp.max(plsc.all_reduce_population_count(m_b))
        return off_a + n_a, off_b + n_b

    off_a, off_b = lax.fori_loop(0, CHE // 16, part_body,
                                 (jnp.int32(0), jnp.int32(0)))
    zpad = jnp.zeros((16,), jnp.int32)
    lane = lax.iota(jnp.int32, 16)
    for q in range(8):
        dpad = HALF + ((q * 16 + lane) % (ACCR - HALF))
        sa_v[pl.ds(off_a + q * 16, 16)] = zpad
        da_v[pl.ds(off_a + q * 16, 16)] = dpad
        sb_v[pl.ds(off_b + q * 16, 16)] = zpad
        db_v[pl.ds(off_b + q * 16, 16)] = dpad

    for p, sl_v, dl_v, cnt in ((0, sa_v, da_v, off_a), (1, sb_v, db_v, off_b)):
        z0 = s * ZPT
        pltpu.sync_copy(zeros_hbm.at[pl.ds(z0, ZPT)],
                        acc_sh.at[pl.ds(z0, ZPT)])
        plsc.subcore_barrier()
        n = (cnt + 127) // 128

        def gath(j, buf, sem):
            pltpu.async_copy(y_hbm.at[sl_v.at[pl.ds(j * 128, 128)]], buf, sem)

        def gwait(j, buf, sem):
            pltpu.make_async_copy(y_hbm.at[sl_v.at[pl.ds(j * 128, 128)]],
                                  buf, sem).wait()

        def scat(j, buf):
            pltpu.sync_copy(buf, acc_sh.at[dl_v.at[pl.ds(j * 128, 128)]],
                            add=True)

        @pl.when(n > 0)
        def _():
            gath(0, ra_v, sem_a)

            def pair(k, carry):
                j = 2 * k
                gath(j + 1, rb_v, sem_b)
                gwait(j, ra_v, sem_a)
                scat(j, ra_v)
                gath(j + 2, ra_v, sem_a)
                gwait(j + 1, rb_v, sem_b)
                scat(j + 1, rb_v)
                return carry

            lax.fori_loop(0, (n - 1) // 2, pair, jnp.int32(0))

            @pl.when(n % 2 == 1)
            def _():
                gwait(n - 1, ra_v, sem_a)
                scat(n - 1, ra_v)

            @pl.when(n % 2 == 0)
            def _():
                gath(n - 1, rb_v, sem_b)
                gwait(n - 2, ra_v, sem_a)
                scat(n - 2, ra_v)
                gwait(n - 1, rb_v, sem_b)
                scat(n - 1, rb_v)

        plsc.subcore_barrier()

        @pl.when(s < 15)
        def _():
            pltpu.sync_copy(
                acc_sh.at[pl.ds(s * 320, 320)],
                out_hbm.at[c, pl.ds(p * HALF + s * 320, 320)])

        @pl.when(s == 15)
        def _():
            pltpu.sync_copy(
                acc_sh.at[pl.ds(4800, 256)],
                out_hbm.at[c, pl.ds(p * HALF + 4800, 256)])

        plsc.subcore_barrier()


def _dinv_of(dp_ref):
    deg = dp_ref[0, :, 0:1] + dp_ref[1, :, 0:1]
    return lax.rsqrt(deg)


def _scale_body(dp_ref, x_ref, y_ref):
    dinv = _dinv_of(dp_ref)
    y_ref[0] = x_ref[:, :FH] * dinv
    y_ref[1] = x_ref[:, FH:] * dinv


def _mid_body(dp_ref, s_ref, y_ref, w1_ref, b1_ref, w2_ref, y2_ref):
    dinv = _dinv_of(dp_ref)
    agg = jnp.concatenate(
        [(s_ref[0] + y_ref[0]) * dinv, (s_ref[1] + y_ref[1]) * dinv], axis=1)
    h = jnp.dot(agg.astype(jnp.bfloat16), w1_ref[...].astype(jnp.bfloat16),
                preferred_element_type=jnp.float32)
    h = jnp.maximum(h + b1_ref[...], 0.0)
    t = jnp.dot(h.astype(jnp.bfloat16), w2_ref[...].astype(jnp.bfloat16),
                preferred_element_type=jnp.float32)
    y2_ref[0] = t[:, :FH] * dinv
    y2_ref[1] = t[:, FH:] * dinv


def _out_body(dp_ref, s_ref, y2_ref, b2_ref, o_ref):
    dinv = _dinv_of(dp_ref)
    o_ref[:, :FH] = (s_ref[0] + y2_ref[0]) * dinv + b2_ref[:, :FH]
    o_ref[:, FH:] = (s_ref[1] + y2_ref[1]) * dinv + b2_ref[:, FH:]


_dp_spec = pl.BlockSpec((NC, BN, 8), lambda i: (0, i, 0))
_half_spec = pl.BlockSpec((NC, BN, FH), lambda i: (0, i, 0))


def _scale(degp, x):
    return pl.pallas_call(
        _scale_body,
        grid=(N // BN,),
        in_specs=[_dp_spec, pl.BlockSpec((BN, F), lambda i: (i, 0))],
        out_specs=pl.BlockSpec((NC, BN, FH), lambda i: (0, i, 0)),
        out_shape=jax.ShapeDtypeStruct((NC, N, FH), jnp.float32),
    )(degp, x)


def _mid(degp, s1, y1, W1, b1, W2):
    return pl.pallas_call(
        _mid_body,
        grid=(N // BN,),
        in_specs=[
            _dp_spec,
            _half_spec,
            _half_spec,
            pl.BlockSpec((F, H), lambda i: (0, 0)),
            pl.BlockSpec((1, H), lambda i: (0, 0)),
            pl.BlockSpec((H, F), lambda i: (0, 0)),
        ],
        out_specs=pl.BlockSpec((NC, BN, FH), lambda i: (0, i, 0)),
        out_shape=jax.ShapeDtypeStruct((NC, N, FH), jnp.float32),
    )(degp, s1, y1, W1, b1, W2)


def _outk(degp, s2, y2, b2):
    return pl.pallas_call(
        _out_body,
        grid=(N // BN,),
        in_specs=[
            _dp_spec,
            _half_spec,
            _half_spec,
            pl.BlockSpec((1, F), lambda i: (0, 0)),
        ],
        out_specs=pl.BlockSpec((BN, F), lambda i: (i, 0)),
        out_shape=jax.ShapeDtypeStruct((N, F), jnp.float32),
    )(degp, s2, y2, b2)


def _prep(edge_index):
    ei = edge_index.astype(jnp.int32)
    src, dst = ei[0], ei[1]
    srcf = jnp.pad(src.reshape(NS, EPT), ((0, 0), (0, CHE - EPT)))
    dstf = jnp.pad(dst.reshape(NS, EPT), ((0, 0), (0, CHE - EPT)),
                   constant_values=1 << 28)
    dd = jnp.pad(dst.reshape(NC, NS, EPTD),
                 ((0, 0), (0, 0), (0, DCH * 128 - EPTD)), constant_values=N)
    dstd = dd.reshape(NC, NS, DCH, 128)
    return srcf, dstf, dstd


def kernel(x, edge_index, W1, b1, W2, b2):
    srcf, dstf, dstd = _prep(edge_index)
    ones8 = jnp.ones((128, 8), jnp.float32)
    zeros8 = jnp.zeros((NPAD, 8), jnp.float32)
    zerosh = jnp.zeros((ACCR, FH), jnp.float32)
    degp = _deg_kernel(dstd, ones8, zeros8)
    y1 = _scale(degp, x)
    s1 = _agg_kernel(y1.reshape(NC * N, FH), srcf, dstf, zerosh)
    y2 = _mid(degp, s1, y1, W1, b1.reshape(1, H), W2)
    s2 = _agg_kernel(y2.reshape(NC * N, FH), srcf, dstf, zerosh)
    return _outk(degp, s2, y2, b2.reshape(1, F))


# speedup vs baseline: 1.0005x; 1.0005x over previous
import dataclasses
import functools

import jax
import jax.numpy as jnp
from jax import lax
from jax.experimental import pallas as pl
from jax.experimental.pallas import tpu as pltpu
from jax.experimental.pallas import tpu_sc as plsc

N = 10000
E = 160000
F = 256
H = 512
FH = 128
NC, NS = 2, 16
NPAD = 10112
HALF = NPAD // 2
ACCR = 5248
ZPT = ACCR // NS
RPT = NPAD // NS
EPT = E // NS
CH = (EPT + 127) // 128
EPTD = E // (NC * NS)
DCH = (EPTD + 127) // 128
BN = 1000

_mesh = plsc.VectorSubcoreMesh(core_axis_name="c", subcore_axis_name="s")

_cp = pltpu.CompilerParams()
if "needs_layout_passes" in pltpu.CompilerParams.__dataclass_fields__:
    _cp = dataclasses.replace(_cp, needs_layout_passes=False)


@functools.partial(
    pl.kernel,
    out_type=jax.ShapeDtypeStruct((NC, NPAD, 8), jnp.float32),
    mesh=_mesh,
    scratch_types=[
        pltpu.VMEM((DCH, 128), jnp.int32),
        pltpu.VMEM((128, 8), jnp.float32),
        pltpu.VMEM_SHARED((NPAD, 8), jnp.float32),
    ],
)
def _deg_kernel(dsti_hbm, ones_hbm, zeros_hbm, out_hbm, idx_v, ones_v, acc_sh):
    c = lax.axis_index("c")
    s = lax.axis_index("s")
    r0 = s * RPT
    pltpu.sync_copy(zeros_hbm.at[pl.ds(r0, RPT)], acc_sh.at[pl.ds(r0, RPT)])
    pltpu.sync_copy(ones_hbm, ones_v)
    pltpu.sync_copy(dsti_hbm.at[c, s], idx_v)
    plsc.subcore_barrier()

    @pl.loop(0, DCH)
    def _(j):
        pltpu.sync_copy(ones_v, acc_sh.at[idx_v.at[j]], add=True)

    plsc.subcore_barrier()
    pltpu.sync_copy(acc_sh.at[pl.ds(r0, RPT)], out_hbm.at[c, pl.ds(r0, RPT)])


CHE = CH * 128


@functools.partial(
    pl.kernel,
    out_type=jax.ShapeDtypeStruct((NC, NPAD, FH), jnp.float32),
    mesh=_mesh,
    scratch_types=[
        pltpu.VMEM((CHE + 128,), jnp.int32),
        pltpu.VMEM((CHE + 128,), jnp.int32),
        pltpu.VMEM((CHE + 128,), jnp.int32),
        pltpu.VMEM((CHE + 128,), jnp.int32),
        pltpu.VMEM((128, FH), jnp.float32),
        pltpu.VMEM((128, FH), jnp.float32),
        pltpu.VMEM_SHARED((ACCR, FH), jnp.float32),
        pltpu.SemaphoreType.DMA,
        pltpu.SemaphoreType.DMA,
    ],
    compiler_params=_cp,
)
def _agg_kernel(y_hbm, srcf_hbm, dstf_hbm, zeros_hbm,
                out_hbm, sa_v, da_v, sb_v, db_v, ra_v, rb_v,
                acc_sh, sem_a, sem_b):
    c = lax.axis_index("c")
    s = lax.axis_index("s")
    coff = c * N

    pltpu.sync_copy(srcf_hbm.at[s], sa_v.at[pl.ds(0, CHE)])
    pltpu.sync_copy(dstf_hbm.at[s], da_v.at[pl.ds(0, CHE)])

    def part_body(i, off):
        off_a, off_b = off
        sv = sa_v[pl.ds(i * 16, 16)] + coff
        dv = da_v[pl.ds(i * 16, 16)]
        m_a = dv < HALF
        m_b = jnp.logical_and(dv >= HALF, dv < N)
        plsc.store_compressed(sa_v.at[pl.ds(off_a, 16)], sv, mask=m_a)
        plsc.store_compressed(da_v.at[pl.ds(off_a, 16)], dv, mask=m_a)
        plsc.store_compressed(sb_v.at[pl.ds(off_b, 16)], sv, mask=m_b)
        plsc.store_compressed(db_v.at[pl.ds(off_b, 16)], dv - HALF, mask=m_b)
        n_a = jnp.max(plsc.all_reduce_population_count(m_a))
        n_b = jnp.max(plsc.all_reduce_population_count(m_b))
        return off_a + n_a, off_b + n_b

    off_a, off_b = lax.fori_loop(0, CHE // 16, part_body,
                                 (jnp.int32(0), jnp.int32(0)))
    zpad = jnp.zeros((16,), jnp.int32)
    lane = lax.iota(jnp.int32, 16)
    for q in range(8):
        dpad = HALF + ((q * 16 + lane) % (ACCR - HALF))
        sa_v[pl.ds(off_a + q * 16, 16)] = zpad
        da_v[pl.ds(off_a + q * 16, 16)] = dpad
        sb_v[pl.ds(off_b + q * 16, 16)] = zpad
        db_v[pl.ds(off_b + q * 16, 16)] = dpad

    for p, sl_v, dl_v, cnt in ((0, sa_v, da_v, off_a), (1, sb_v, db_v, off_b)):
        z0 = s * ZPT
        pltpu.sync_copy(zeros_hbm.at[pl.ds(z0, ZPT)],
                        acc_sh.at[pl.ds(z0, ZPT)])
        plsc.subcore_barrier()
        n = (cnt + 127) // 128

        def gath(j, buf, sem):
            pltpu.async_copy(y_hbm.at[sl_v.at[pl.ds(j * 128, 128)]], buf, sem)

        def gwait(j, buf, sem):
            pltpu.make_async_copy(y_hbm.at[sl_v.at[pl.ds(j * 128, 128)]],
                                  buf, sem).wait()

        def scat(j, buf):
            pltpu.sync_copy(buf, acc_sh.at[dl_v.at[pl.ds(j * 128, 128)]],
                            add=True)

        @pl.when(n > 0)
        def _():
            gath(0, ra_v, sem_a)

            def pair(k, carry):
                j = 2 * k
                gath(j + 1, rb_v, sem_b)
                gwait(j, ra_v, sem_a)
                scat(j, ra_v)
                gath(j + 2, ra_v, sem_a)
                gwait(j + 1, rb_v, sem_b)
                scat(j + 1, rb_v)
                return carry

            lax.fori_loop(0, (n - 1) // 2, pair, jnp.int32(0))

            @pl.when(n % 2 == 1)
            def _():
                gwait(n - 1, ra_v, sem_a)
                scat(n - 1, ra_v)

            @pl.when(n % 2 == 0)
            def _():
                gath(n - 1, rb_v, sem_b)
                gwait(n - 2, ra_v, sem_a)
                scat(n - 2, ra_v)
                gwait(n - 1, rb_v, sem_b)
                scat(n - 1, rb_v)

        plsc.subcore_barrier()

        @pl.when(s < 15)
        def _():
            pltpu.sync_copy(
                acc_sh.at[pl.ds(s * 320, 320)],
                out_hbm.at[c, pl.ds(p * HALF + s * 320, 320)])

        @pl.when(s == 15)
        def _():
            pltpu.sync_copy(
                acc_sh.at[pl.ds(4800, 256)],
                out_hbm.at[c, pl.ds(p * HALF + 4800, 256)])

        plsc.subcore_barrier()


def _dinv_of(dp_ref):
    deg = dp_ref[0, :, 0:1] + dp_ref[1, :, 0:1]
    return lax.rsqrt(deg)


def _scale_body(dp_ref, x_ref, y_ref):
    dinv = _dinv_of(dp_ref)
    y_ref[0] = x_ref[:, :FH] * dinv
    y_ref[1] = x_ref[:, FH:] * dinv


def _mid_body(dp_ref, s_ref, y_ref, w1_ref, b1_ref, w2_ref, y2_ref):
    dinv = _dinv_of(dp_ref)
    agg = jnp.concatenate(
        [(s_ref[0] + y_ref[0]) * dinv, (s_ref[1] + y_ref[1]) * dinv], axis=1)
    h = jnp.dot(agg, w1_ref[...], preferred_element_type=jnp.float32)
    h = jnp.maximum(h + b1_ref[...], 0.0)
    t = jnp.dot(h, w2_ref[...], preferred_element_type=jnp.float32)
    y2_ref[0] = t[:, :FH] * dinv
    y2_ref[1] = t[:, FH:] * dinv


def _out_body(dp_ref, s_ref, y2_ref, b2_ref, o_ref):
    dinv = _dinv_of(dp_ref)
    o_ref[:, :FH] = (s_ref[0] + y2_ref[0]) * dinv + b2_ref[:, :FH]
    o_ref[:, FH:] = (s_ref[1] + y2_ref[1]) * dinv + b2_ref[:, FH:]


_dp_spec = pl.BlockSpec((NC, BN, 8), lambda i: (0, i, 0))
_half_spec = pl.BlockSpec((NC, BN, FH), lambda i: (0, i, 0))


def _scale(degp, x):
    return pl.pallas_call(
        _scale_body,
        grid=(N // BN,),
        in_specs=[_dp_spec, pl.BlockSpec((BN, F), lambda i: (i, 0))],
        out_specs=pl.BlockSpec((NC, BN, FH), lambda i: (0, i, 0)),
        out_shape=jax.ShapeDtypeStruct((NC, N, FH), jnp.float32),
    )(degp, x)


def _mid(degp, s1, y1, W1, b1, W2):
    return pl.pallas_call(
        _mid_body,
        grid=(N // BN,),
        in_specs=[
            _dp_spec,
            _half_spec,
            _half_spec,
            pl.BlockSpec((F, H), lambda i: (0, 0)),
            pl.BlockSpec((1, H), lambda i: (0, 0)),
            pl.BlockSpec((H, F), lambda i: (0, 0)),
        ],
        out_specs=pl.BlockSpec((NC, BN, FH), lambda i: (0, i, 0)),
        out_shape=jax.ShapeDtypeStruct((NC, N, FH), jnp.float32),
    )(degp, s1, y1, W1, b1, W2)


def _outk(degp, s2, y2, b2):
    return pl.pallas_call(
        _out_body,
        grid=(N // BN,),
        in_specs=[
            _dp_spec,
            _half_spec,
            _half_spec,
            pl.BlockSpec((1, F), lambda i: (0, 0)),
        ],
        out_specs=pl.BlockSpec((BN, F), lambda i: (i, 0)),
        out_shape=jax.ShapeDtypeStruct((N, F), jnp.float32),
    )(degp, s2, y2, b2)


def _prep(edge_index):
    ei = edge_index.astype(jnp.int32)
    src, dst = ei[0], ei[1]
    srcf = jnp.pad(src.reshape(NS, EPT), ((0, 0), (0, CHE - EPT)))
    dstf = jnp.pad(dst.reshape(NS, EPT), ((0, 0), (0, CHE - EPT)),
                   constant_values=1 << 28)
    dd = jnp.pad(dst.reshape(NC, NS, EPTD),
                 ((0, 0), (0, 0), (0, DCH * 128 - EPTD)), constant_values=N)
    dstd = dd.reshape(NC, NS, DCH, 128)
    return srcf, dstf, dstd


def kernel(x, edge_index, W1, b1, W2, b2):
    srcf, dstf, dstd = _prep(edge_index)
    ones8 = jnp.ones((128, 8), jnp.float32)
    zeros8 = jnp.zeros((NPAD, 8), jnp.float32)
    zerosh = jnp.zeros((ACCR, FH), jnp.float32)
    degp = _deg_kernel(dstd, ones8, zeros8)
    y1 = _scale(degp, x)
    s1 = _agg_kernel(y1.reshape(NC * N, FH), srcf, dstf, zerosh)
    y2 = _mid(degp, s1, y1, W1, b1.reshape(1, H), W2)
    s2 = _agg_kernel(y2.reshape(NC * N, FH), srcf, dstf, zerosh)
    return _outk(degp, s2, y2, b2.reshape(1, F))
